# baseline (speedup 1.0000x reference)
import jax
import jax.numpy as jnp
from jax import lax
from jax.experimental import pallas as pl

_C = 81
_THRESH = 0.5
_V0 = 0.1
_V1 = 0.2


def _smooth_l1(x):
    ax = jnp.abs(x)
    return jnp.where(ax < 1.0, 0.5 * x * x, ax - 0.5)


def _match_kernel(tg_ref, pri_ref, lx_ref, ly_ref, lw_ref, lh_ref,
                  conf_t_ref, lossl_ref, nump_ref, *, n_obj, nchunks, ccols):
    total = nchunks * ccols
    tgv = tg_ref[0]
    tx1 = tgv[:, 0:1]
    ty1 = tgv[:, 1:2]
    tx2 = tgv[:, 2:3]
    ty2 = tgv[:, 3:4]
    lab = tgv[:, 4:5]
    area_a = (tx2 - tx1) * (ty2 - ty1)
    oio = lax.broadcasted_iota(jnp.int32, (n_obj, ccols), 0)
    lane = lax.broadcasted_iota(jnp.int32, (1, ccols), 1)

    def chunk_ov(ci):
        px = pri_ref[0, ci:ci + 1, :]
        py = pri_ref[1, ci:ci + 1, :]
        pw = pri_ref[2, ci:ci + 1, :]
        ph = pri_ref[3, ci:ci + 1, :]
        xmin = px - pw / 2.0
        ymin = py - ph / 2.0
        xmax = px + pw / 2.0
        ymax = py + ph / 2.0
        area_b = (xmax - xmin) * (ymax - ymin)
        iw = jnp.clip(jnp.minimum(tx2, xmax) - jnp.maximum(tx1, xmin),
                      0.0, None)
        ih = jnp.clip(jnp.minimum(ty2, ymax) - jnp.maximum(ty1, ymin),
                      0.0, None)
        inter = iw * ih
        return inter / (area_a + area_b - inter)

    bpm = None
    bpi = None
    for ci in range(nchunks):
        ov = chunk_ov(ci)
        cm = jnp.max(ov, axis=1, keepdims=True)
        fidx = lane + ci * ccols
        cidx = jnp.min(jnp.where(ov == cm, fidx, total), axis=1,
                       keepdims=True)
        if ci == 0:
            bpm, bpi = cm, cidx
        else:
            take = cm > bpm
            bpi = jnp.where(take, cidx, bpi)
            bpm = jnp.where(take, cm, bpm)

    conf_rows = []
    m_rows = [[] for _ in range(4)]
    for ci in range(nchunks):
        ov = chunk_ov(ci)
        btov = jnp.max(ov, axis=0, keepdims=True)
        btidx = jnp.min(jnp.where(ov == btov, oio, n_obj), axis=0,
                        keepdims=True)
        fidx = lane + ci * ccols
        hit = fidx == bpi
        sel_o = jnp.max(jnp.where(hit, oio, -1), axis=0, keepdims=True)
        forced = sel_o >= 0
        btidx = jnp.where(forced, sel_o, btidx)
        btov = jnp.where(forced, 2.0, btov)
        selm = oio == btidx

        def gat(col):
            return jnp.sum(jnp.where(selm, col, 0.0), axis=0, keepdims=True)

        conf = jnp.where(btov < _THRESH, 0.0, gat(lab))
        conf_rows.append(conf.astype(jnp.int32))
        for j, col in enumerate((tx1, ty1, tx2, ty2)):
            m_rows[j].append(gat(col))

    conf_i = jnp.concatenate(conf_rows, axis=0)
    mx1 = jnp.concatenate(m_rows[0], axis=0)
    my1 = jnp.concatenate(m_rows[1], axis=0)
    mx2 = jnp.concatenate(m_rows[2], axis=0)
    my2 = jnp.concatenate(m_rows[3], axis=0)
    conf_t_ref[0] = conf_i

    px = pri_ref[0]
    py = pri_ref[1]
    pw = pri_ref[2]
    ph = pri_ref[3]
    g_cx = ((mx1 + mx2) / 2.0 - px) / (_V0 * pw)
    g_cy = ((my1 + my2) / 2.0 - py) / (_V0 * ph)
    g_w = jnp.log((mx2 - mx1) / pw) / _V1
    g_h = jnp.log((my2 - my1) / ph) / _V1

    posf = (conf_i > 0).astype(jnp.float32)
    l = (_smooth_l1(lx_ref[0] - g_cx) + _smooth_l1(ly_ref[0] - g_cy)
         + _smooth_l1(lw_ref[0] - g_w) + _smooth_l1(lh_ref[0] - g_h))
    lossl_ref[0] = jnp.full((1, 1), jnp.sum(l * posf), jnp.float32)
    nump_ref[0] = jnp.full((1, 1), jnp.sum(posf), jnp.float32)


def _ce_kernel(conf_ref, t_ref, ce_ref, *, rows):
    x = conf_ref[...]
    t = t_ref[...]
    s = jnp.sum(jnp.exp(x), axis=1, keepdims=True)
    cio = lax.broadcasted_iota(jnp.int32, (rows, _C), 1)
    g = jnp.max(jnp.where(cio == t, x, -jnp.inf), axis=1, keepdims=True)
    ce_ref[...] = jnp.log(s) - g


def _mine_kernel(ce_ref, t_ref, lossl_ref, nump_ref, out_ref, *, num_priors,
                 negpos_ratio):
    ce = ce_ref[...]
    t = t_ref[...]
    pos = t > 0
    posf = pos.astype(jnp.float32)
    np_i = jnp.sum(posf, axis=1, keepdims=True)
    k = jnp.minimum(negpos_ratio * np_i, float(num_priors - 1))
    pos_ce = jnp.sum(ce * posf)
    masked = jnp.maximum(jnp.where(pos, 0.0, ce), 0.0)
    mv = lax.bitcast_convert_type(masked, jnp.int32)
    lo = jnp.zeros(np_i.shape, jnp.int32)
    for bit in range(30, -1, -1):
        cand = lo | jnp.int32(1 << bit)
        cnt = jnp.sum((mv >= cand).astype(jnp.float32), axis=1, keepdims=True)
        lo = jnp.where(cnt >= k, cand, lo)
    tval = lax.bitcast_convert_type(lo, jnp.float32)
    gt = masked > tval
    cnt_gt = jnp.sum(gt.astype(jnp.float32), axis=1, keepdims=True)
    sum_gt = jnp.sum(jnp.where(gt, masked, 0.0), axis=1, keepdims=True)
    topk = sum_gt + (k - cnt_gt) * tval
    oio = lax.broadcasted_iota(jnp.int32, (1, 8), 1)
    row = jnp.where(oio == 0, pos_ce + jnp.sum(topk), 0.0)
    row = jnp.where(oio == 1, jnp.sum(nump_ref[...]), row)
    row = jnp.where(oio == 2, jnp.sum(lossl_ref[...]), row)
    out_ref[...] = row


def kernel(loc_data, conf_data, priors, targets):
    num, num_priors, _ = loc_data.shape
    n_obj = targets.shape[1]
    nchunks = 8
    ccols = num_priors // nchunks

    pri4 = priors.T.reshape(4, nchunks, ccols)
    locp = [loc_data[:, :, j].reshape(num, nchunks, ccols) for j in range(4)]

    loc_spec = pl.BlockSpec((1, nchunks, ccols), lambda i: (i, 0, 0))
    conf_t, lossl, nump = pl.pallas_call(
        lambda *refs: _match_kernel(*refs, n_obj=n_obj,
                                    nchunks=nchunks, ccols=ccols),
        grid=(num,),
        in_specs=[
            pl.BlockSpec((1, n_obj, 5), lambda i: (i, 0, 0)),
            pl.BlockSpec((4, nchunks, ccols), lambda i: (0, 0, 0)),
            loc_spec, loc_spec, loc_spec, loc_spec,
        ],
        out_specs=[
            pl.BlockSpec((1, nchunks, ccols), lambda i: (i, 0, 0)),
            pl.BlockSpec((1, 1, 1), lambda i: (i, 0, 0)),
            pl.BlockSpec((1, 1, 1), lambda i: (i, 0, 0)),
        ],
        out_shape=[
            jax.ShapeDtypeStruct((num, nchunks, ccols), jnp.int32),
            jax.ShapeDtypeStruct((num, 1, 1), jnp.float32),
            jax.ShapeDtypeStruct((num, 1, 1), jnp.float32),
        ],
    )(targets, pri4, *locp)

    n_anchor = num * num_priors
    blk = 8192
    conf2 = conf_data.reshape(n_anchor, _C)
    t2 = conf_t.reshape(n_anchor, 1)
    ce = pl.pallas_call(
        lambda a, b, c: _ce_kernel(a, b, c, rows=blk),
        grid=(n_anchor // blk,),
        in_specs=[
            pl.BlockSpec((blk, _C), lambda i: (i, 0)),
            pl.BlockSpec((blk, 1), lambda i: (i, 0)),
        ],
        out_specs=pl.BlockSpec((blk, 1), lambda i: (i, 0)),
        out_shape=jax.ShapeDtypeStruct((n_anchor, 1), jnp.float32),
    )(conf2, t2)

    out = pl.pallas_call(
        lambda a, b, c, d, e: _mine_kernel(a, b, c, d, e,
                                           num_priors=num_priors,
                                           negpos_ratio=3.0),
        in_specs=[
            pl.BlockSpec((num, num_priors), lambda: (0, 0)),
            pl.BlockSpec((num, num_priors), lambda: (0, 0)),
            pl.BlockSpec((num, 1, 1), lambda: (0, 0, 0)),
            pl.BlockSpec((num, 1, 1), lambda: (0, 0, 0)),
        ],
        out_specs=pl.BlockSpec((1, 8), lambda: (0, 0)),
        out_shape=jax.ShapeDtypeStruct((1, 8), jnp.float32),
    )(ce.reshape(num, num_priors), conf_t.reshape(num, num_priors),
      lossl, nump)

    loss_c = out[0, 0]
    n = out[0, 1]
    loss_l = out[0, 2]
    return (loss_l / n, loss_c / n)


# speedup vs baseline: 1.0312x; 1.0312x over previous
import jax
import jax.numpy as jnp
from jax import lax
from jax.experimental import pallas as pl

_C = 81
_THRESH = 0.5
_V0 = 0.1
_V1 = 0.2


def _smooth_l1(x):
    ax = jnp.abs(x)
    return jnp.where(ax < 1.0, 0.5 * x * x, ax - 0.5)


def _match_kernel(tg_ref, pri_ref, loc_ref, conf_t_ref, lossl_ref, nump_ref,
                  *, n_obj, nchunks, ccols):
    total = nchunks * ccols
    tgv = tg_ref[0]
    tx1 = tgv[:, 0:1]
    ty1 = tgv[:, 1:2]
    tx2 = tgv[:, 2:3]
    ty2 = tgv[:, 3:4]
    lab = tgv[:, 4:5]
    area_a = (tx2 - tx1) * (ty2 - ty1)
    oio = lax.broadcasted_iota(jnp.int32, (n_obj, ccols), 0)
    lane = lax.broadcasted_iota(jnp.int32, (1, ccols), 1)

    def chunk_ov(ci):
        px = pri_ref[0, ci:ci + 1, :]
        py = pri_ref[1, ci:ci + 1, :]
        pw = pri_ref[2, ci:ci + 1, :]
        ph = pri_ref[3, ci:ci + 1, :]
        xmin = px - pw / 2.0
        ymin = py - ph / 2.0
        xmax = px + pw / 2.0
        ymax = py + ph / 2.0
        area_b = (xmax - xmin) * (ymax - ymin)
        iw = jnp.clip(jnp.minimum(tx2, xmax) - jnp.maximum(tx1, xmin),
                      0.0, None)
        ih = jnp.clip(jnp.minimum(ty2, ymax) - jnp.maximum(ty1, ymin),
                      0.0, None)
        inter = iw * ih
        return inter / (area_a + area_b - inter)

    bpm = None
    bpi = None
    for ci in range(nchunks):
        ov = chunk_ov(ci)
        cm = jnp.max(ov, axis=1, keepdims=True)
        fidx = lane + ci * ccols
        cidx = jnp.min(jnp.where(ov == cm, fidx, total), axis=1,
                       keepdims=True)
        if ci == 0:
            bpm, bpi = cm, cidx
        else:
            take = cm > bpm
            bpi = jnp.where(take, cidx, bpi)
            bpm = jnp.where(take, cm, bpm)

    conf_rows = []
    m_rows = [[] for _ in range(4)]
    for ci in range(nchunks):
        ov = chunk_ov(ci)
        btov = jnp.max(ov, axis=0, keepdims=True)
        btidx = jnp.min(jnp.where(ov == btov, oio, n_obj), axis=0,
                        keepdims=True)
        fidx = lane + ci * ccols
        hit = fidx == bpi
        sel_o = jnp.max(jnp.where(hit, oio, -1), axis=0, keepdims=True)
        forced = sel_o >= 0
        btidx = jnp.where(forced, sel_o, btidx)
        btov = jnp.where(forced, 2.0, btov)
        selm = oio == btidx

        def gat(col):
            return jnp.sum(jnp.where(selm, col, 0.0), axis=0, keepdims=True)

        conf = jnp.where(btov < _THRESH, 0.0, gat(lab))
        conf_rows.append(conf.astype(jnp.int32))
        for j, col in enumerate((tx1, ty1, tx2, ty2)):
            m_rows[j].append(gat(col))

    conf_i = jnp.concatenate(conf_rows, axis=0)
    mx1 = jnp.concatenate(m_rows[0], axis=0)
    my1 = jnp.concatenate(m_rows[1], axis=0)
    mx2 = jnp.concatenate(m_rows[2], axis=0)
    my2 = jnp.concatenate(m_rows[3], axis=0)
    conf_t_ref[0] = conf_i

    px = pri_ref[0]
    py = pri_ref[1]
    pw = pri_ref[2]
    ph = pri_ref[3]
    g_cx = ((mx1 + mx2) / 2.0 - px) / (_V0 * pw)
    g_cy = ((my1 + my2) / 2.0 - py) / (_V0 * ph)
    g_w = jnp.log((mx2 - mx1) / pw) / _V1
    g_h = jnp.log((my2 - my1) / ph) / _V1

    posf = (conf_i > 0).astype(jnp.float32)
    l = (_smooth_l1(loc_ref[0, 0] - g_cx) + _smooth_l1(loc_ref[0, 1] - g_cy)
         + _smooth_l1(loc_ref[0, 2] - g_w) + _smooth_l1(loc_ref[0, 3] - g_h))
    lossl_ref[0] = jnp.full((1, 1), jnp.sum(l * posf), jnp.float32)
    nump_ref[0] = jnp.full((1, 1), jnp.sum(posf), jnp.float32)


def _ce_kernel(conf_ref, t_ref, ce_ref, *, rows):
    x = conf_ref[...]
    t = t_ref[...]
    s = jnp.sum(jnp.exp(x), axis=1, keepdims=True)
    cio = lax.broadcasted_iota(jnp.int32, (rows, _C), 1)
    g = jnp.max(jnp.where(cio == t, x, -jnp.inf), axis=1, keepdims=True)
    ce_ref[...] = jnp.log(s) - g


def _mine_kernel(ce_ref, t_ref, lossl_ref, nump_ref, out_ref, *, num_priors,
                 negpos_ratio):
    ce = ce_ref[...]
    t = t_ref[...]
    pos = t > 0
    posf = pos.astype(jnp.float32)
    np_i = jnp.sum(posf, axis=1, keepdims=True)
    k = jnp.minimum(negpos_ratio * np_i, float(num_priors - 1))
    pos_ce = jnp.sum(ce * posf)
    masked = jnp.maximum(jnp.where(pos, 0.0, ce), 0.0)
    mv = lax.bitcast_convert_type(masked, jnp.int32)
    lo = jnp.zeros(np_i.shape, jnp.int32)
    for bit in range(30, -1, -1):
        cand = lo | jnp.int32(1 << bit)
        cnt = jnp.sum((mv >= cand).astype(jnp.float32), axis=1, keepdims=True)
        lo = jnp.where(cnt >= k, cand, lo)
    tval = lax.bitcast_convert_type(lo, jnp.float32)
    gt = masked > tval
    cnt_gt = jnp.sum(gt.astype(jnp.float32), axis=1, keepdims=True)
    sum_gt = jnp.sum(jnp.where(gt, masked, 0.0), axis=1, keepdims=True)
    topk = sum_gt + (k - cnt_gt) * tval
    oio = lax.broadcasted_iota(jnp.int32, (1, 8), 1)
    row = jnp.where(oio == 0, pos_ce + jnp.sum(topk), 0.0)
    row = jnp.where(oio == 1, jnp.sum(nump_ref[...]), row)
    row = jnp.where(oio == 2, jnp.sum(lossl_ref[...]), row)
    out_ref[...] = row


def kernel(loc_data, conf_data, priors, targets):
    num, num_priors, _ = loc_data.shape
    n_obj = targets.shape[1]
    nchunks = 8
    ccols = num_priors // nchunks

    pri4 = priors.T.reshape(4, nchunks, ccols)
    loc4 = loc_data.transpose(0, 2, 1).reshape(num, 4, nchunks, ccols)

    conf_t, lossl, nump = pl.pallas_call(
        lambda a, b, c, d, e, f: _match_kernel(a, b, c, d, e, f, n_obj=n_obj,
                                               nchunks=nchunks, ccols=ccols),
        grid=(num,),
        in_specs=[
            pl.BlockSpec((1, n_obj, 5), lambda i: (i, 0, 0)),
            pl.BlockSpec((4, nchunks, ccols), lambda i: (0, 0, 0)),
            pl.BlockSpec((1, 4, nchunks, ccols), lambda i: (i, 0, 0, 0)),
        ],
        out_specs=[
            pl.BlockSpec((1, nchunks, ccols), lambda i: (i, 0, 0)),
            pl.BlockSpec((1, 1, 1), lambda i: (i, 0, 0)),
            pl.BlockSpec((1, 1, 1), lambda i: (i, 0, 0)),
        ],
        out_shape=[
            jax.ShapeDtypeStruct((num, nchunks, ccols), jnp.int32),
            jax.ShapeDtypeStruct((num, 1, 1), jnp.float32),
            jax.ShapeDtypeStruct((num, 1, 1), jnp.float32),
        ],
    )(targets, pri4, loc4)

    n_anchor = num * num_priors
    blk = 8192
    conf2 = conf_data.reshape(n_anchor, _C)
    t2 = conf_t.reshape(n_anchor, 1)
    ce = pl.pallas_call(
        lambda a, b, c: _ce_kernel(a, b, c, rows=blk),
        grid=(n_anchor // blk,),
        in_specs=[
            pl.BlockSpec((blk, _C), lambda i: (i, 0)),
            pl.BlockSpec((blk, 1), lambda i: (i, 0)),
        ],
        out_specs=pl.BlockSpec((blk, 1), lambda i: (i, 0)),
        out_shape=jax.ShapeDtypeStruct((n_anchor, 1), jnp.float32),
    )(conf2, t2)

    out = pl.pallas_call(
        lambda a, b, c, d, e: _mine_kernel(a, b, c, d, e,
                                           num_priors=num_priors,
                                           negpos_ratio=3.0),
        in_specs=[
            pl.BlockSpec((num, num_priors), lambda: (0, 0)),
            pl.BlockSpec((num, num_priors), lambda: (0, 0)),
            pl.BlockSpec((num, 1, 1), lambda: (0, 0, 0)),
            pl.BlockSpec((num, 1, 1), lambda: (0, 0, 0)),
        ],
        out_specs=pl.BlockSpec((1, 8), lambda: (0, 0)),
        out_shape=jax.ShapeDtypeStruct((1, 8), jnp.float32),
    )(ce.reshape(num, num_priors), conf_t.reshape(num, num_priors),
      lossl, nump)

    loss_c = out[0, 0]
    n = out[0, 1]
    loss_l = out[0, 2]
    return (loss_l / n, loss_c / n)
